# Initial kernel scaffold; baseline (speedup 1.0000x reference)
import jax
import jax.numpy as jnp
from jax.experimental import pallas as pl


def kernel(x, pos_table):
    raise NotImplementedError("write your pallas kernel here")


# speedup vs baseline: 3.0114x; 3.0114x over previous
import jax
import jax.numpy as jnp
from jax.experimental import pallas as pl


def _embed_kernel(pos_ref, out_ref):
    out_ref[...] = pos_ref[...]


def kernel(x, pos_table):
    seq_len = x.shape[1]
    d_model = pos_table.shape[1]
    block_rows = 1024
    grid = seq_len // block_rows
    return pl.pallas_call(
        _embed_kernel,
        out_shape=jax.ShapeDtypeStruct((seq_len, d_model), pos_table.dtype),
        grid=(grid,),
        in_specs=[pl.BlockSpec((block_rows, d_model), lambda i: (i, 0))],
        out_specs=pl.BlockSpec((block_rows, d_model), lambda i: (i, 0)),
    )(pos_table)


# speedup vs baseline: 3.2437x; 1.0771x over previous
import jax
import jax.numpy as jnp
from jax.experimental import pallas as pl


def _embed_kernel(pos_ref, out_ref):
    out_ref[...] = pos_ref[...]


def kernel(x, pos_table):
    seq_len = x.shape[1]
    d_model = pos_table.shape[1]
    block_rows = 2048
    grid = seq_len // block_rows
    return pl.pallas_call(
        _embed_kernel,
        out_shape=jax.ShapeDtypeStruct((seq_len, d_model), pos_table.dtype),
        grid=(grid,),
        in_specs=[pl.BlockSpec((block_rows, d_model), lambda i: (i, 0))],
        out_specs=pl.BlockSpec((block_rows, d_model), lambda i: (i, 0)),
    )(pos_table)


# speedup vs baseline: 3.2585x; 1.0046x over previous
import jax
import jax.numpy as jnp
from jax.experimental import pallas as pl
from jax.experimental.pallas import tpu as pltpu


def _embed_kernel(pos_ref, out_ref):
    out_ref[...] = pos_ref[...]


def kernel(x, pos_table):
    seq_len = x.shape[1]
    d_model = pos_table.shape[1]
    block_rows = 2048
    grid = seq_len // block_rows
    return pl.pallas_call(
        _embed_kernel,
        out_shape=jax.ShapeDtypeStruct((seq_len, d_model), pos_table.dtype),
        grid=(grid,),
        in_specs=[pl.BlockSpec((block_rows, d_model), lambda i: (i, 0))],
        out_specs=pl.BlockSpec((block_rows, d_model), lambda i: (i, 0)),
        compiler_params=pltpu.CompilerParams(
            dimension_semantics=("parallel",),
        ),
    )(pos_table)
